# Initial kernel scaffold; baseline (speedup 1.0000x reference)
import jax
import jax.numpy as jnp
from jax.experimental import pallas as pl


def kernel(nlist, atype_ext):
    raise NotImplementedError("write your pallas kernel here")


# speedup vs baseline: 797.8764x; 797.8764x over previous
import functools

import jax
import jax.numpy as jnp
from jax import lax
from jax.experimental import pallas as pl
from jax.experimental.pallas import tpu as pltpu
from jax.experimental.pallas import tpu_sc as plsc

_NC = 2
_NS = 16
_NW = _NC * _NS
_L = 16


def _body(nlist_hbm, atype_hbm, out_hbm, table_v, in_v, out_v, *,
          nloc, nnei, nall, rows_pw, rows_pb):
    nblocks = rows_pw // rows_pb
    chunk = rows_pb * nnei

    wid = lax.axis_index("s") * _NC + lax.axis_index("c")
    row0 = wid * rows_pw
    f = row0 // nloc
    lrow0 = row0 - f * nloc

    pltpu.sync_copy(atype_hbm.at[f], table_v)

    def block(b, _):
        off = lrow0 * nnei + b * chunk
        pltpu.sync_copy(nlist_hbm.at[f, pl.ds(off, chunk)], in_v)

        def row(r, _):
            i = lrow0 + b * rows_pb + r
            ti = plsc.load_gather(
                table_v, [jnp.broadcast_to(i, (_L,)).astype(jnp.int32)])
            rv = jnp.where(ti < 4, ti ^ 1, jnp.int32(-5))
            for j in range(nnei // _L):
                o = r * nnei + j * _L
                nl = in_v[pl.ds(o, _L)]
                tj = plsc.load_gather(table_v, [jnp.maximum(nl, 0)])
                out_v[pl.ds(o, _L)] = ((tj != rv) | (nl < 0)).astype(jnp.int32)
            return 0

        lax.fori_loop(0, rows_pb, row, 0)
        pltpu.sync_copy(out_v, out_hbm.at[f, pl.ds(off, chunk)])
        return 0

    lax.fori_loop(0, nblocks, block, 0)


@jax.jit
def kernel(nlist, atype_ext):
    nf, nloc, nnei = nlist.shape
    nall = atype_ext.shape[1]
    assert (nf * nloc) % _NW == 0
    rows_pw = (nf * nloc) // _NW
    assert nloc % rows_pw == 0
    assert nnei % _L == 0
    rows_pb = min(rows_pw, 128)
    assert rows_pw % rows_pb == 0

    nlist_flat = nlist.reshape(nf, nloc * nnei)
    mesh = plsc.VectorSubcoreMesh(
        core_axis_name="c", subcore_axis_name="s",
        num_cores=_NC, num_subcores=_NS)
    body = functools.partial(
        _body, nloc=nloc, nnei=nnei, nall=nall,
        rows_pw=rows_pw, rows_pb=rows_pb)
    out = pl.kernel(
        body,
        out_type=jax.ShapeDtypeStruct((nf, nloc * nnei), jnp.int32),
        mesh=mesh,
        compiler_params=pltpu.CompilerParams(needs_layout_passes=False),
        scratch_types=[
            pltpu.VMEM((nall,), jnp.int32),
            pltpu.VMEM((rows_pb * nnei,), jnp.int32),
            pltpu.VMEM((rows_pb * nnei,), jnp.int32),
        ],
    )(nlist_flat, atype_ext)
    return out.reshape(nf, nloc, nnei)
